# Initial kernel scaffold; baseline (speedup 1.0000x reference)
import functools

import jax
import jax.numpy as jnp
from jax import lax
from jax.experimental import pallas as pl
from jax.experimental.pallas import tpu as pltpu
from jax.experimental.pallas import tpu_sc as plsc

N_NODES = 10000
N_EDGES = 320000
D_FEAT = 128

NC = 2
NS = 16
NW = NC * NS
C = 128
K = 80
NBUF = 4
NF = D_FEAT // 16
ROWS_PT = N_NODES // NS


def _sc_graph_conv(x, sidx_p, tidx_p, w_p):
    mesh = plsc.VectorSubcoreMesh(core_axis_name="c", subcore_axis_name="s")

    @functools.partial(
        pl.kernel,
        out_type=(jax.ShapeDtypeStruct((N_NODES, D_FEAT), jnp.float32),) * 2,
        mesh=mesh,
        scratch_types=(
            [
                pltpu.VMEM((K, C), jnp.int32),
                pltpu.VMEM((K, C), jnp.int32),
                pltpu.VMEM((K, C), jnp.float32),
            ]
            + [pltpu.VMEM((C, D_FEAT), jnp.float32) for _ in range(NBUF)]
            + [pltpu.VMEM_SHARED((N_NODES, D_FEAT), jnp.float32)]
            + [pltpu.SemaphoreType.DMA for _ in range(2 * NBUF)]
        ),
    )
    def body(x_hbm, sidx_hbm, tidx_hbm, w_hbm, out0, out1,
             sidx_v, tidx_v, w_v, b0, b1, b2, b3, acc,
             g0, g1, g2, g3, s0, s1, s2, s3):
        bufs = (b0, b1, b2, b3)
        gsem = (g0, g1, g2, g3)
        ssem = (s0, s1, s2, s3)
        cid = lax.axis_index("c")
        sid = lax.axis_index("s")
        wid = cid * NS + sid

        pltpu.sync_copy(sidx_hbm.at[wid], sidx_v)
        pltpu.sync_copy(tidx_hbm.at[wid], tidx_v)
        pltpu.sync_copy(w_hbm.at[wid], w_v)

        zbuf = bufs[0]

        def zrow(i, carry):
            for f in range(NF):
                zbuf[i, pl.ds(f * 16, 16)] = jnp.zeros((16,), jnp.float32)
            return carry

        lax.fori_loop(0, C, zrow, 0)
        base = sid * ROWS_PT
        nfull = ROWS_PT // C
        rem = ROWS_PT - nfull * C
        for q in range(nfull):
            pltpu.sync_copy(zbuf, acc.at[pl.ds(base + q * C, C)])
        if rem:
            pltpu.sync_copy(zbuf.at[pl.ds(0, rem)],
                            acc.at[pl.ds(base + nfull * C, rem)])
        plsc.subcore_barrier()

        def g_start(jj, b):
            pltpu.async_copy(x_hbm.at[sidx_v.at[jj]], bufs[b], gsem[b])

        def g_wait(jj, b):
            pltpu.make_async_copy(x_hbm.at[sidx_v.at[jj]], bufs[b],
                                  gsem[b]).wait()

        def s_start(jj, b):
            pltpu.async_copy(bufs[b], acc.at[tidx_v.at[jj]], ssem[b],
                             add=True)

        def s_wait(jj, b):
            pltpu.make_async_copy(bufs[b], acc.at[tidx_v.at[jj]],
                                  ssem[b]).wait()

        def scale(jj, b):
            buf = bufs[b]

            def grp(g, carry):
                wv = w_v[jj, pl.ds(g * 16, 16)]
                for e in range(16):
                    ws = wv[e]
                    r = g * 16 + e
                    for f in range(NF):
                        buf[r, pl.ds(f * 16, 16)] = (
                            buf[r, pl.ds(f * 16, 16)] * ws)
                return carry

            lax.fori_loop(0, C // 16, grp, 0)

        g_start(0, 0)

        def step(i, carry):
            j = i * NBUF
            for b in range(NBUF):
                jj = j + b
                bn = (b + 1) % NBUF

                @pl.when(jj >= NBUF - 1)
                def _():
                    s_wait(jj - (NBUF - 1), bn)

                @pl.when(jj + 1 < K)
                def _():
                    g_start(jj + 1, bn)

                g_wait(jj, b)
                scale(jj, b)
                s_start(jj, b)
            return carry

        lax.fori_loop(0, K // NBUF, step, 0)
        for jj in range(K - NBUF + 1, K):
            s_wait(jj, jj % NBUF)

        plsc.subcore_barrier()

        @pl.when(cid == 0)
        def _():
            pltpu.sync_copy(acc.at[pl.ds(base, ROWS_PT)],
                            out0.at[pl.ds(base, ROWS_PT)])

        @pl.when(cid == 1)
        def _():
            pltpu.sync_copy(acc.at[pl.ds(base, ROWS_PT)],
                            out1.at[pl.ds(base, ROWS_PT)])

    return body(x, sidx_p, tidx_p, w_p)


def _tc_add(a, b):
    def add_body(a_ref, b_ref, o_ref):
        o_ref[...] = a_ref[...] + b_ref[...]

    return pl.pallas_call(
        add_body,
        out_shape=jax.ShapeDtypeStruct((N_NODES, D_FEAT), jnp.float32),
        grid=(10,),
        in_specs=[pl.BlockSpec((N_NODES // 10, D_FEAT), lambda i: (i, 0))] * 2,
        out_specs=pl.BlockSpec((N_NODES // 10, D_FEAT), lambda i: (i, 0)),
    )(a, b)


def kernel(input, eidx, enorm, esgn):
    sidx = eidx[0].astype(jnp.int32)
    tidx = eidx[1].astype(jnp.int32)
    w = enorm * esgn
    pad = NW * K * C - N_EDGES
    pad_nodes = jnp.arange(pad, dtype=jnp.int32) % N_NODES
    sidx_p = jnp.concatenate([sidx, pad_nodes]).reshape(NW, K, C)
    tidx_p = jnp.concatenate([tidx, pad_nodes]).reshape(NW, K, C)
    w_p = jnp.concatenate([w, jnp.zeros((pad,), jnp.float32)]).reshape(NW, K, C)
    p0, p1 = _sc_graph_conv(input, sidx_p, tidx_p, w_p)
    return _tc_add(p0, p1)


# speedup vs baseline: 5.0810x; 5.0810x over previous
import functools

import jax
import jax.numpy as jnp
from jax import lax
from jax.experimental import pallas as pl
from jax.experimental.pallas import tpu as pltpu
from jax.experimental.pallas import tpu_sc as plsc

N_NODES = 10000
N_EDGES = 320000
D_FEAT = 128
DH = D_FEAT // 2

NC = 2
NS = 16
C = 96
K = 212
EPT = K * C
NBUF = 4
NFH = DH // 16
ROWS_PT = 640
N_PAD = NS * ROWS_PT
LAST_ROWS = N_NODES - (NS - 1) * ROWS_PT


def _sc_graph_conv(xlo, xhi, sidx_p, tidx_p, w_p):
    mesh = plsc.VectorSubcoreMesh(core_axis_name="c", subcore_axis_name="s",
                                  num_cores=NC, num_subcores=NS)

    @functools.partial(
        pl.kernel,
        out_type=(jax.ShapeDtypeStruct((N_NODES, DH), jnp.float32),) * 2,
        mesh=mesh,
        compiler_params=pltpu.CompilerParams(use_tc_tiling_on_sc=False),
        scratch_types=(
            [
                pltpu.VMEM((K, C), jnp.int32),
                pltpu.VMEM((K, C), jnp.int32),
                pltpu.VMEM((K, C), jnp.float32),
            ]
            + [pltpu.VMEM((C, DH), jnp.float32) for _ in range(NBUF)]
            + [pltpu.VMEM_SHARED((N_PAD, DH), jnp.float32)]
            + [pltpu.SemaphoreType.DMA for _ in range(2 * NBUF)]
        ),
    )
    def body(xlo_hbm, xhi_hbm, sidx_hbm, tidx_hbm, w_hbm, olo, ohi,
             sidx_v, tidx_v, w_v, b0, b1, b2, b3, acc,
             g0, g1, g2, g3, s0, s1, s2, s3):
        bufs = (b0, b1, b2, b3)
        gsem = (g0, g1, g2, g3)
        ssem = (s0, s1, s2, s3)
        cid = lax.axis_index("c")
        sid = lax.axis_index("s")

        pltpu.sync_copy(sidx_hbm.at[sid], sidx_v)
        pltpu.sync_copy(tidx_hbm.at[sid], tidx_v)
        pltpu.sync_copy(w_hbm.at[sid], w_v)

        zbuf = bufs[0]

        def zrow(i, carry):
            for f in range(NFH):
                zbuf[i, pl.ds(f * 16, 16)] = jnp.zeros((16,), jnp.float32)
            return carry

        lax.fori_loop(0, C, zrow, 0)
        base = sid * ROWS_PT
        nfull = ROWS_PT // C
        rem = ROWS_PT - nfull * C
        for q in range(nfull):
            pltpu.sync_copy(zbuf, acc.at[pl.ds(base + q * C, C)])
        if rem:
            pltpu.sync_copy(zbuf.at[pl.ds(0, rem)],
                            acc.at[pl.ds(base + nfull * C, rem)])
        plsc.subcore_barrier()

        def g_start(jj, b):
            @pl.when(cid == 0)
            def _():
                pltpu.async_copy(xlo_hbm.at[sidx_v.at[jj]], bufs[b], gsem[b])

            @pl.when(cid == 1)
            def _():
                pltpu.async_copy(xhi_hbm.at[sidx_v.at[jj]], bufs[b], gsem[b])

        def g_wait(jj, b):
            pltpu.make_async_copy(xlo_hbm.at[sidx_v.at[jj]], bufs[b],
                                  gsem[b]).wait()

        def s_start(jj, b):
            pltpu.async_copy(bufs[b], acc.at[tidx_v.at[jj]], ssem[b],
                             add=True)

        def s_wait(jj, b):
            pltpu.make_async_copy(bufs[b], acc.at[tidx_v.at[jj]],
                                  ssem[b]).wait()

        def scale(jj, b):
            buf = bufs[b]

            def grp(g, carry):
                wv = w_v[jj, pl.ds(g * 16, 16)]
                for e in range(16):
                    ws = wv[e]
                    r = g * 16 + e
                    for f in range(NFH):
                        buf[r, pl.ds(f * 16, 16)] = (
                            buf[r, pl.ds(f * 16, 16)] * ws)
                return carry

            lax.fori_loop(0, C // 16, grp, 0)

        g_start(0, 0)

        def step(i, carry):
            j = i * NBUF
            for b in range(NBUF):
                jj = j + b
                bn = (b + 1) % NBUF

                @pl.when(jj >= NBUF - 1)
                def _():
                    s_wait(jj - (NBUF - 1), bn)

                @pl.when(jj + 1 < K)
                def _():
                    g_start(jj + 1, bn)

                g_wait(jj, b)
                scale(jj, b)
                s_start(jj, b)
            return carry

        lax.fori_loop(0, K // NBUF, step, 0)
        for jj in range(K - NBUF + 1, K):
            s_wait(jj, jj % NBUF)

        plsc.subcore_barrier()

        for out_ref, my_cid in ((olo, 0), (ohi, 1)):
            @pl.when(jnp.logical_and(cid == my_cid, sid < NS - 1))
            def _(out_ref=out_ref):
                pltpu.sync_copy(acc.at[pl.ds(base, ROWS_PT)],
                                out_ref.at[pl.ds(base, ROWS_PT)])

            @pl.when(jnp.logical_and(cid == my_cid, sid == NS - 1))
            def _(out_ref=out_ref):
                pltpu.sync_copy(acc.at[pl.ds(base, LAST_ROWS)],
                                out_ref.at[pl.ds(base, LAST_ROWS)])

    return body(xlo, xhi, sidx_p, tidx_p, w_p)


def kernel(input, eidx, enorm, esgn):
    sidx = eidx[0].astype(jnp.int32)
    tidx = eidx[1].astype(jnp.int32)
    w = enorm * esgn
    pad = NS * EPT - N_EDGES
    pad_nodes = jnp.arange(pad, dtype=jnp.int32) % N_NODES
    sidx_p = jnp.concatenate([sidx, pad_nodes]).reshape(NS, K, C)
    tidx_p = jnp.concatenate([tidx, pad_nodes]).reshape(NS, K, C)
    w_p = jnp.concatenate([w, jnp.zeros((pad,), jnp.float32)]).reshape(NS, K, C)
    olo, ohi = _sc_graph_conv(input[:, :DH], input[:, DH:],
                              sidx_p, tidx_p, w_p)
    return jnp.concatenate([olo, ohi], axis=1)


# speedup vs baseline: 10.4542x; 2.0575x over previous
import functools

import jax
import jax.numpy as jnp
from jax import lax
from jax.experimental import pallas as pl
from jax.experimental.pallas import tpu as pltpu
from jax.experimental.pallas import tpu_sc as plsc

N_NODES = 10000
N_EDGES = 320000
D_FEAT = 128
DH = D_FEAT // 2

NC = 2
NS = 16
C = 96
K = 212
EPT = K * C
NBUF = 4
NFH = DH // 16
ROWS_PT = 640
N_PAD = NS * ROWS_PT
LAST_ROWS = N_NODES - (NS - 1) * ROWS_PT


def _sc_graph_conv(xlo, xhi, sidx_p, tidx_p, w_p):
    mesh = plsc.VectorSubcoreMesh(core_axis_name="c", subcore_axis_name="s",
                                  num_cores=NC, num_subcores=NS)

    @functools.partial(
        pl.kernel,
        out_type=(jax.ShapeDtypeStruct((N_NODES, DH), jnp.float32),) * 2,
        mesh=mesh,
        compiler_params=pltpu.CompilerParams(use_tc_tiling_on_sc=False),
        scratch_types=(
            [
                pltpu.VMEM((K, C), jnp.int32),
                pltpu.VMEM((K, C), jnp.int32),
                pltpu.VMEM((K, C), jnp.float32),
            ]
            + [pltpu.VMEM((C, DH), jnp.float32) for _ in range(NBUF)]
            + [pltpu.VMEM_SHARED((N_PAD, DH), jnp.float32)]
            + [pltpu.SemaphoreType.DMA for _ in range(2 * NBUF)]
        ),
    )
    def body(xlo_hbm, xhi_hbm, sidx_hbm, tidx_hbm, w_hbm, olo, ohi,
             sidx_v, tidx_v, w_v, b0, b1, b2, b3, acc,
             g0, g1, g2, g3, s0, s1, s2, s3):
        bufs = (b0, b1, b2, b3)
        gsem = (g0, g1, g2, g3)
        ssem = (s0, s1, s2, s3)
        cid = lax.axis_index("c")
        sid = lax.axis_index("s")

        pltpu.sync_copy(sidx_hbm.at[sid], sidx_v)
        pltpu.sync_copy(tidx_hbm.at[sid], tidx_v)
        pltpu.sync_copy(w_hbm.at[sid], w_v)

        zbuf = bufs[0]

        def zrow(i, carry):
            for f in range(NFH):
                zbuf[i, pl.ds(f * 16, 16)] = jnp.zeros((16,), jnp.float32)
            return carry

        lax.fori_loop(0, C, zrow, 0)
        base = sid * ROWS_PT
        nfull = ROWS_PT // C
        rem = ROWS_PT - nfull * C
        for q in range(nfull):
            pltpu.sync_copy(zbuf, acc.at[pl.ds(base + q * C, C)])
        if rem:
            pltpu.sync_copy(zbuf.at[pl.ds(0, rem)],
                            acc.at[pl.ds(base + nfull * C, rem)])
        plsc.subcore_barrier()

        def g_start(jj, b):
            @pl.when(cid == 0)
            def _():
                pltpu.async_copy(xlo_hbm.at[sidx_v.at[jj]], bufs[b], gsem[b])

            @pl.when(cid == 1)
            def _():
                pltpu.async_copy(xhi_hbm.at[sidx_v.at[jj]], bufs[b], gsem[b])

        def g_wait(jj, b):
            pltpu.make_async_copy(xlo_hbm.at[sidx_v.at[jj]], bufs[b],
                                  gsem[b]).wait()

        def s_start(jj, b):
            pltpu.async_copy(bufs[b], acc.at[tidx_v.at[jj]], ssem[b],
                             add=True)

        def s_wait(jj, b):
            pltpu.make_async_copy(bufs[b], acc.at[tidx_v.at[jj]],
                                  ssem[b]).wait()

        def scale(jj, b):
            buf = bufs[b]

            @plsc.parallel_loop(0, C // 16, unroll=2)
            def grp(g):
                wv = w_v[jj, pl.ds(g * 16, 16)]
                for e in range(16):
                    ws = wv[e]
                    r = g * 16 + e
                    for f in range(NFH):
                        buf[r, pl.ds(f * 16, 16)] = (
                            buf[r, pl.ds(f * 16, 16)] * ws)

        g_start(0, 0)

        def step(i, carry):
            j = i * NBUF
            for b in range(NBUF):
                jj = j + b
                bn = (b + 1) % NBUF

                @pl.when(jj >= NBUF - 1)
                def _():
                    s_wait(jj - (NBUF - 1), bn)

                @pl.when(jj + 1 < K)
                def _():
                    g_start(jj + 1, bn)

                g_wait(jj, b)
                scale(jj, b)
                s_start(jj, b)
            return carry

        lax.fori_loop(0, K // NBUF, step, 0)
        for jj in range(K - NBUF + 1, K):
            s_wait(jj, jj % NBUF)

        plsc.subcore_barrier()

        for out_ref, my_cid in ((olo, 0), (ohi, 1)):
            @pl.when(jnp.logical_and(cid == my_cid, sid < NS - 1))
            def _(out_ref=out_ref):
                pltpu.sync_copy(acc.at[pl.ds(base, ROWS_PT)],
                                out_ref.at[pl.ds(base, ROWS_PT)])

            @pl.when(jnp.logical_and(cid == my_cid, sid == NS - 1))
            def _(out_ref=out_ref):
                pltpu.sync_copy(acc.at[pl.ds(base, LAST_ROWS)],
                                out_ref.at[pl.ds(base, LAST_ROWS)])

    return body(xlo, xhi, sidx_p, tidx_p, w_p)


def kernel(input, eidx, enorm, esgn):
    sidx = eidx[0].astype(jnp.int32)
    tidx = eidx[1].astype(jnp.int32)
    w = enorm * esgn
    pad = NS * EPT - N_EDGES
    pad_nodes = jnp.arange(pad, dtype=jnp.int32) % N_NODES
    sidx_p = jnp.concatenate([sidx, pad_nodes]).reshape(NS, K, C)
    tidx_p = jnp.concatenate([tidx, pad_nodes]).reshape(NS, K, C)
    w_p = jnp.concatenate([w, jnp.zeros((pad,), jnp.float32)]).reshape(NS, K, C)
    olo, ohi = _sc_graph_conv(input[:, :DH], input[:, DH:],
                              sidx_p, tidx_p, w_p)
    return jnp.concatenate([olo, ohi], axis=1)


# speedup vs baseline: 11.8500x; 1.1335x over previous
import functools

import jax
import jax.numpy as jnp
from jax import lax
from jax.experimental import pallas as pl
from jax.experimental.pallas import tpu as pltpu
from jax.experimental.pallas import tpu_sc as plsc

N_NODES = 10000
N_EDGES = 320000
D_FEAT = 128
DH = D_FEAT // 2

NC = 2
NS = 16
C = 96
K = 212
EPT = K * C
NBUF = 4
NFH = DH // 16
ROWS_PT = 640
N_PAD = NS * ROWS_PT
LAST_ROWS = N_NODES - (NS - 1) * ROWS_PT


def _sc_graph_conv(xlo, xhi, sidx_p, tidx_p, w_p):
    mesh = plsc.VectorSubcoreMesh(core_axis_name="c", subcore_axis_name="s",
                                  num_cores=NC, num_subcores=NS)

    @functools.partial(
        pl.kernel,
        out_type=(jax.ShapeDtypeStruct((N_NODES, DH), jnp.float32),) * 2,
        mesh=mesh,
        compiler_params=pltpu.CompilerParams(use_tc_tiling_on_sc=False),
        scratch_types=(
            [
                pltpu.VMEM((K, C), jnp.int32),
                pltpu.VMEM((K, C), jnp.int32),
                pltpu.VMEM((K, C), jnp.float32),
            ]
            + [pltpu.VMEM((C, DH), jnp.float32) for _ in range(NBUF)]
            + [pltpu.VMEM_SHARED((N_PAD, DH), jnp.float32)]
            + [pltpu.SemaphoreType.DMA for _ in range(2 * NBUF)]
        ),
    )
    def body(xlo_hbm, xhi_hbm, sidx_hbm, tidx_hbm, w_hbm, olo, ohi,
             sidx_v, tidx_v, w_v, b0, b1, b2, b3, acc,
             g0, g1, g2, g3, s0, s1, s2, s3):
        bufs = (b0, b1, b2, b3)
        gsem = (g0, g1, g2, g3)
        ssem = (s0, s1, s2, s3)
        cid = lax.axis_index("c")
        sid = lax.axis_index("s")

        pltpu.sync_copy(sidx_hbm.at[sid], sidx_v)
        pltpu.sync_copy(tidx_hbm.at[sid], tidx_v)
        pltpu.sync_copy(w_hbm.at[sid], w_v)

        zbuf = bufs[0]

        def zrow(i, carry):
            for f in range(NFH):
                zbuf[i, pl.ds(f * 16, 16)] = jnp.zeros((16,), jnp.float32)
            return carry

        lax.fori_loop(0, C, zrow, 0)
        base = sid * ROWS_PT
        nfull = ROWS_PT // C
        rem = ROWS_PT - nfull * C
        for q in range(nfull):
            pltpu.sync_copy(zbuf, acc.at[pl.ds(base + q * C, C)])
        if rem:
            pltpu.sync_copy(zbuf.at[pl.ds(0, rem)],
                            acc.at[pl.ds(base + nfull * C, rem)])
        plsc.subcore_barrier()

        def g_start(jj, b):
            @pl.when(cid == 0)
            def _():
                pltpu.async_copy(xlo_hbm.at[sidx_v.at[jj]], bufs[b], gsem[b])

            @pl.when(cid == 1)
            def _():
                pltpu.async_copy(xhi_hbm.at[sidx_v.at[jj]], bufs[b], gsem[b])

        def g_wait(jj, b):
            pltpu.make_async_copy(xlo_hbm.at[sidx_v.at[jj]], bufs[b],
                                  gsem[b]).wait()

        def s_start(jj, b):
            pltpu.async_copy(bufs[b], acc.at[tidx_v.at[jj]], ssem[b],
                             add=True)

        def s_wait(jj, b):
            pltpu.make_async_copy(bufs[b], acc.at[tidx_v.at[jj]],
                                  ssem[b]).wait()

        def scale(jj, b):
            buf = bufs[b]

            @plsc.parallel_loop(0, C // 16, unroll=2)
            def grp(g):
                wv = w_v[jj, pl.ds(g * 16, 16)]
                for e in range(16):
                    ws = wv[e]
                    r = g * 16 + e
                    for f in range(NFH):
                        buf[r, pl.ds(f * 16, 16)] = (
                            buf[r, pl.ds(f * 16, 16)] * ws)

        g_start(0, 0)
        g_start(1, 1)

        def step(i, carry):
            j = i * NBUF
            for b in range(NBUF):
                jj = j + b
                b2 = (b + 2) % NBUF

                @pl.when(jj >= NBUF - 2)
                def _():
                    s_wait(jj - (NBUF - 2), b2)

                @pl.when(jj + 2 < K)
                def _():
                    g_start(jj + 2, b2)

                g_wait(jj, b)
                scale(jj, b)
                s_start(jj, b)
            return carry

        lax.fori_loop(0, K // NBUF, step, 0)
        for jj in range(K - NBUF + 2, K):
            s_wait(jj, jj % NBUF)

        plsc.subcore_barrier()

        for out_ref, my_cid in ((olo, 0), (ohi, 1)):
            @pl.when(jnp.logical_and(cid == my_cid, sid < NS - 1))
            def _(out_ref=out_ref):
                pltpu.sync_copy(acc.at[pl.ds(base, ROWS_PT)],
                                out_ref.at[pl.ds(base, ROWS_PT)])

            @pl.when(jnp.logical_and(cid == my_cid, sid == NS - 1))
            def _(out_ref=out_ref):
                pltpu.sync_copy(acc.at[pl.ds(base, LAST_ROWS)],
                                out_ref.at[pl.ds(base, LAST_ROWS)])

    return body(xlo, xhi, sidx_p, tidx_p, w_p)


def kernel(input, eidx, enorm, esgn):
    sidx = eidx[0].astype(jnp.int32)
    tidx = eidx[1].astype(jnp.int32)
    w = enorm * esgn
    pad = NS * EPT - N_EDGES
    pad_nodes = jnp.arange(pad, dtype=jnp.int32) % N_NODES
    sidx_p = jnp.concatenate([sidx, pad_nodes]).reshape(NS, K, C)
    tidx_p = jnp.concatenate([tidx, pad_nodes]).reshape(NS, K, C)
    w_p = jnp.concatenate([w, jnp.zeros((pad,), jnp.float32)]).reshape(NS, K, C)
    olo, ohi = _sc_graph_conv(input[:, :DH], input[:, DH:],
                              sidx_p, tidx_p, w_p)
    return jnp.concatenate([olo, ohi], axis=1)


# speedup vs baseline: 12.7836x; 1.0788x over previous
import functools

import jax
import jax.numpy as jnp
from jax import lax
from jax.experimental import pallas as pl
from jax.experimental.pallas import tpu as pltpu
from jax.experimental.pallas import tpu_sc as plsc

N_NODES = 10000
N_EDGES = 320000
D_FEAT = 128
DH = D_FEAT // 2

NC = 2
NS = 16
C = 96
K = 212
EPT = K * C
NBUF = 4
NFH = DH // 16
ROWS_PT = 640
N_PAD = NS * ROWS_PT
LAST_ROWS = N_NODES - (NS - 1) * ROWS_PT


def _sc_graph_conv(xlo, xhi, sidx_p, tidx_p, w_p):
    mesh = plsc.VectorSubcoreMesh(core_axis_name="c", subcore_axis_name="s",
                                  num_cores=NC, num_subcores=NS)

    @functools.partial(
        pl.kernel,
        out_type=jax.ShapeDtypeStruct((N_NODES, D_FEAT), jnp.float32),
        mesh=mesh,
        compiler_params=pltpu.CompilerParams(use_tc_tiling_on_sc=False),
        scratch_types=(
            [
                pltpu.VMEM((K, C), jnp.int32),
                pltpu.VMEM((K, C), jnp.int32),
                pltpu.VMEM((K, C), jnp.float32),
            ]
            + [pltpu.VMEM((C, DH), jnp.float32) for _ in range(NBUF)]
            + [pltpu.VMEM_SHARED((N_PAD, DH), jnp.float32)]
            + [pltpu.SemaphoreType.DMA for _ in range(2 * NBUF)]
        ),
    )
    def body(xlo_hbm, xhi_hbm, sidx_hbm, tidx_hbm, w_hbm, out,
             sidx_v, tidx_v, w_v, b0, b1, b2, b3, acc,
             g0, g1, g2, g3, s0, s1, s2, s3):
        bufs = (b0, b1, b2, b3)
        gsem = (g0, g1, g2, g3)
        ssem = (s0, s1, s2, s3)
        cid = lax.axis_index("c")
        sid = lax.axis_index("s")

        pltpu.sync_copy(sidx_hbm.at[sid], sidx_v)
        pltpu.sync_copy(tidx_hbm.at[sid], tidx_v)
        pltpu.sync_copy(w_hbm.at[sid], w_v)

        zbuf = bufs[0]

        def zrow(i, carry):
            for f in range(NFH):
                zbuf[i, pl.ds(f * 16, 16)] = jnp.zeros((16,), jnp.float32)
            return carry

        lax.fori_loop(0, C, zrow, 0)
        base = sid * ROWS_PT
        nfull = ROWS_PT // C
        rem = ROWS_PT - nfull * C
        for q in range(nfull):
            pltpu.sync_copy(zbuf, acc.at[pl.ds(base + q * C, C)])
        if rem:
            pltpu.sync_copy(zbuf.at[pl.ds(0, rem)],
                            acc.at[pl.ds(base + nfull * C, rem)])
        plsc.subcore_barrier()

        def g_start(jj, b):
            @pl.when(cid == 0)
            def _():
                pltpu.async_copy(xlo_hbm.at[sidx_v.at[jj]], bufs[b], gsem[b])

            @pl.when(cid == 1)
            def _():
                pltpu.async_copy(xhi_hbm.at[sidx_v.at[jj]], bufs[b], gsem[b])

        def g_wait(jj, b):
            pltpu.make_async_copy(xlo_hbm.at[sidx_v.at[jj]], bufs[b],
                                  gsem[b]).wait()

        def s_start(jj, b):
            pltpu.async_copy(bufs[b], acc.at[tidx_v.at[jj]], ssem[b],
                             add=True)

        def s_wait(jj, b):
            pltpu.make_async_copy(bufs[b], acc.at[tidx_v.at[jj]],
                                  ssem[b]).wait()

        def scale(jj, b):
            buf = bufs[b]

            @plsc.parallel_loop(0, C // 16, unroll=2)
            def grp(g):
                wv = w_v[jj, pl.ds(g * 16, 16)]
                for e in range(16):
                    ws = wv[e]
                    r = g * 16 + e
                    for f in range(NFH):
                        buf[r, pl.ds(f * 16, 16)] = (
                            buf[r, pl.ds(f * 16, 16)] * ws)

        g_start(0, 0)
        g_start(1, 1)

        def step(i, carry):
            j = i * NBUF
            for b in range(NBUF):
                jj = j + b
                b2 = (b + 2) % NBUF

                @pl.when(jj >= NBUF - 2)
                def _():
                    s_wait(jj - (NBUF - 2), b2)

                @pl.when(jj + 2 < K)
                def _():
                    g_start(jj + 2, b2)

                g_wait(jj, b)
                scale(jj, b)
                s_start(jj, b)
            return carry

        lax.fori_loop(0, K // NBUF, step, 0)
        for jj in range(K - NBUF + 2, K):
            s_wait(jj, jj % NBUF)

        plsc.subcore_barrier()

        col = cid * DH

        @pl.when(sid < NS - 1)
        def _():
            pltpu.sync_copy(acc.at[pl.ds(base, ROWS_PT)],
                            out.at[pl.ds(base, ROWS_PT), pl.ds(col, DH)])

        @pl.when(sid == NS - 1)
        def _():
            pltpu.sync_copy(acc.at[pl.ds(base, LAST_ROWS)],
                            out.at[pl.ds(base, LAST_ROWS), pl.ds(col, DH)])

    return body(xlo, xhi, sidx_p, tidx_p, w_p)


def kernel(input, eidx, enorm, esgn):
    sidx = eidx[0].astype(jnp.int32)
    tidx = eidx[1].astype(jnp.int32)
    w = enorm * esgn
    pad = NS * EPT - N_EDGES
    pad_nodes = jnp.arange(pad, dtype=jnp.int32) % N_NODES
    sidx_p = jnp.concatenate([sidx, pad_nodes]).reshape(NS, K, C)
    tidx_p = jnp.concatenate([tidx, pad_nodes]).reshape(NS, K, C)
    w_p = jnp.concatenate([w, jnp.zeros((pad,), jnp.float32)]).reshape(NS, K, C)
    return _sc_graph_conv(input[:, :DH], input[:, DH:],
                          sidx_p, tidx_p, w_p)
